# Initial kernel scaffold; baseline (speedup 1.0000x reference)
import jax
import jax.numpy as jnp
from jax.experimental import pallas as pl


def kernel(past_traj, past_traj_rel, past_traj_timestamp_mask, is_predictable, same_scene_mask, W_pos, b_pos, W_soc, b_soc, W_ih, W_hh, b_ih, b_hh, h0, c0, W_pred, b_pred):
    raise NotImplementedError("write your pallas kernel here")


# speedup vs baseline: 30.3662x; 30.3662x over previous
import jax
import jax.numpy as jnp
from jax.experimental import pallas as pl

N = 512
T_ENC = 8
T_DEC = 12
EMBED = 64
HID = 128


def _lane_cumsum_excl(x):
    rows = x.shape[0]
    inc = x
    k = 1
    while k < N:
        z = jnp.zeros((rows, k), jnp.int32)
        inc = inc + jnp.concatenate([z, inc[:, :N - k]], axis=1)
        k *= 2
    return inc - x


def _social_lstm_body(trajT_ref, trajC_ref, relC_ref, scene_col_ref,
                      scene_row_ref, W_posT_ref, b_pos_ref, W_socT_ref,
                      b_soc_ref, W_ihT_ref, W_hhT_ref, b_g_ref, h0_ref,
                      c0_ref, W_predT_ref, b_pred_ref, out_ref, traj_ref):
    f32 = jnp.float32
    bf16 = jnp.bfloat16
    rows_i = jax.lax.broadcasted_iota(jnp.int32, (N, N), 0)
    cols_i = jax.lax.broadcasted_iota(jnp.int32, (N, N), 1)
    noteye = rows_i != cols_i
    eye_f = jnp.where(noteye, 0.0, 1.0).astype(f32)
    sn = (scene_col_ref[...] == scene_row_ref[...]) & noteye

    W_posT = W_posT_ref[...].astype(bf16)
    b_pos = b_pos_ref[...]
    W_socT = W_socT_ref[...].astype(bf16)
    b_soc = b_soc_ref[...]
    W_ihT = W_ihT_ref[...].astype(bf16)
    W_hhT = W_hhT_ref[...].astype(bf16)
    b_g = b_g_ref[...]

    def step(ht, ct, pcol, prow2, rel_t):
        rx = prow2[0:1, :] - pcol[:, 0:1]
        ry = prow2[1:2, :] - pcol[:, 1:2]
        within = (rx < 0.99) & (rx > -0.99) & (ry < 0.99) & (ry > -0.99)
        mb = within & sn
        ax = (rx + 1.0) >= 1.0
        ay = (ry + 1.0) >= 1.0
        q = []
        for g in range(4):
            qx = ax if (g // 2) else jnp.logical_not(ax)
            qy = ay if (g % 2) else jnp.logical_not(ay)
            q.append(mb & qx & qy)
        m_all = jnp.concatenate(q, axis=0)
        m_int = m_all.astype(jnp.int32)
        rank = _lane_cumsum_excl(m_int)
        rid = jnp.where(m_all, rank, -1)
        gmax = jnp.max(rid) + 1

        h1 = ht.astype(bf16)
        r1 = ht - h1.astype(f32)
        h2 = r1.astype(bf16)
        r2 = r1 - h2.astype(f32)
        h3 = r2.astype(bf16)
        h_cat = jnp.concatenate([h1, h2, h3], axis=1)

        def group_body(r, acc):
            mr = jnp.where(rid == r, 1.0, 0.0).astype(bf16)
            d = jax.lax.dot(mr, h_cat, preferred_element_type=f32)
            return acc + ((d[:, :HID] + d[:, HID:2 * HID])
                          + d[:, 2 * HID:])

        acc0 = jnp.zeros((4 * N, HID), f32)
        acc = jax.lax.fori_loop(0, gmax, group_body, acc0)
        pooled_flat = jnp.concatenate(
            [acc[0:N], acc[N:2 * N], acc[2 * N:3 * N], acc[3 * N:]], axis=1)

        at = jnp.maximum(
            jax.lax.dot(pooled_flat.astype(bf16), W_socT,
                        preferred_element_type=f32) + b_soc, 0.0)
        et = jnp.maximum(
            jax.lax.dot(rel_t.astype(bf16), W_posT,
                        preferred_element_type=f32) + b_pos, 0.0)
        x = jnp.concatenate([et, at], axis=1)
        gates = (jax.lax.dot(x.astype(bf16), W_ihT,
                             preferred_element_type=f32)
                 + jax.lax.dot(ht.astype(bf16), W_hhT,
                               preferred_element_type=f32)
                 + b_g)
        i_g = jax.nn.sigmoid(gates[:, 0:HID])
        f_g = jax.nn.sigmoid(gates[:, HID:2 * HID])
        g_g = jnp.tanh(gates[:, 2 * HID:3 * HID])
        o_g = jax.nn.sigmoid(gates[:, 3 * HID:4 * HID])
        c2 = f_g * ct + i_g * g_g
        h2_ = o_g * jnp.tanh(c2)
        return h2_, c2

    ht0 = jnp.broadcast_to(h0_ref[...], (N, HID))
    ct0 = jnp.broadcast_to(c0_ref[...], (N, HID))

    def enc_body(t, carry):
        ht, ct = carry
        return step(ht, ct, trajC_ref[t], trajT_ref[t], relC_ref[t])

    ht, ct = jax.lax.fori_loop(0, T_ENC, enc_body, (ht0, ct0))

    W_predT = W_predT_ref[...].astype(bf16)
    b_pred = b_pred_ref[...]

    def dec_body(t, carry):
        ht, ct, ppoint = carry
        out = jax.lax.dot(ht.astype(bf16), W_predT,
                          preferred_element_type=f32) + b_pred
        out_ref[t] = out
        prel = out[:, 0:2]
        ppoint = ppoint + prel
        traj_ref[t] = ppoint
        rowx = jnp.sum(eye_f * ppoint[:, 0:1], axis=0, keepdims=True)
        rowy = jnp.sum(eye_f * ppoint[:, 1:2], axis=0, keepdims=True)
        prow2 = jnp.concatenate([rowx, rowy], axis=0)
        ht, ct = step(ht, ct, ppoint, prow2, prel)
        return ht, ct, ppoint

    jax.lax.fori_loop(0, T_DEC, dec_body, (ht, ct, trajC_ref[T_ENC - 1]))


def kernel(past_traj, past_traj_rel, past_traj_timestamp_mask, is_predictable,
           same_scene_mask, W_pos, b_pos, W_soc, b_soc, W_ih, W_hh, b_ih,
           b_hh, h0, c0, W_pred, b_pred):
    f32 = jnp.float32
    trajT = jnp.transpose(past_traj, (1, 2, 0))
    trajC = jnp.transpose(past_traj, (1, 0, 2))
    relC = jnp.transpose(past_traj_rel, (1, 0, 2))
    scene_col = same_scene_mask
    scene_row = jnp.reshape(same_scene_mask, (1, N))
    out, traj = pl.pallas_call(
        _social_lstm_body,
        out_shape=[
            jax.ShapeDtypeStruct((T_DEC, N, 5), f32),
            jax.ShapeDtypeStruct((T_DEC, N, 2), f32),
        ],
    )(trajT, trajC, relC, scene_col, scene_row,
      W_pos.T, b_pos[None, :], W_soc.T, b_soc[None, :],
      W_ih.T, W_hh.T, (b_ih + b_hh)[None, :], h0[None, :], c0[None, :],
      W_pred.T, b_pred[None, :])
    return out.transpose(1, 0, 2), traj.transpose(1, 0, 2)


# speedup vs baseline: 40.5201x; 1.3344x over previous
import jax
import jax.numpy as jnp
from jax.experimental import pallas as pl

N = 512
T_ENC = 8
T_DEC = 12
EMBED = 64
HID = 128


def _social_lstm_body(trajT_ref, trajC_ref, relC_ref, scene_col_ref,
                      scene_row_ref, W_posT_ref, b_pos_ref, W_socT_ref,
                      b_soc_ref, W_ihT_ref, W_hhT_ref, b_g_ref, h0_ref,
                      c0_ref, W_predT_ref, b_pred_ref, out_ref, traj_ref):
    f32 = jnp.float32
    bf16 = jnp.bfloat16
    rows_i = jax.lax.broadcasted_iota(jnp.int32, (N, N), 0)
    cols_i = jax.lax.broadcasted_iota(jnp.int32, (N, N), 1)
    noteye = rows_i != cols_i
    eye_f = jnp.where(noteye, 0.0, 1.0).astype(f32)
    tri = jnp.where(rows_i < cols_i, 1.0, 0.0).astype(bf16)
    sn = (scene_col_ref[...] == scene_row_ref[...]) & noteye

    W_posT = W_posT_ref[...].astype(bf16)
    b_pos = b_pos_ref[...]
    W_socT = W_socT_ref[...].astype(bf16)
    b_soc = b_soc_ref[...]
    W_ihT = W_ihT_ref[...].astype(bf16)
    W_hhT = W_hhT_ref[...].astype(bf16)
    b_g = b_g_ref[...]

    def step(ht, ct, pcol, prow2, rel_t):
        rx = prow2[0:1, :] - pcol[:, 0:1]
        ry = prow2[1:2, :] - pcol[:, 1:2]
        within = (rx < 0.99) & (rx > -0.99) & (ry < 0.99) & (ry > -0.99)
        mb = within & sn
        ax = (rx + 1.0) >= 1.0
        ay = (ry + 1.0) >= 1.0
        q = []
        for g in range(4):
            qx = ax if (g // 2) else jnp.logical_not(ax)
            qy = ay if (g % 2) else jnp.logical_not(ay)
            q.append(mb & qx & qy)
        m_all = jnp.concatenate(q, axis=0)
        m_bf = jnp.where(m_all, 1.0, 0.0).astype(bf16)
        rank = jax.lax.dot(m_bf, tri, preferred_element_type=f32)
        rid = jnp.where(m_all, rank, -1.0)
        gmax = (jnp.max(rid) + 1.0).astype(jnp.int32)

        h1 = ht.astype(bf16)
        r1 = ht - h1.astype(f32)
        h2 = r1.astype(bf16)
        r2 = r1 - h2.astype(f32)
        h3 = r2.astype(bf16)
        h_cat = jnp.concatenate([h1, h2, h3], axis=1)

        def group_body(r, acc):
            rf = r.astype(f32)
            mr = jnp.where(rid == rf, 1.0, 0.0).astype(bf16)
            d = jax.lax.dot(mr, h_cat, preferred_element_type=f32)
            return acc + ((d[:, :HID] + d[:, HID:2 * HID])
                          + d[:, 2 * HID:])

        acc0 = jnp.zeros((4 * N, HID), f32)
        acc = jax.lax.fori_loop(0, gmax, group_body, acc0)
        pooled_flat = jnp.concatenate(
            [acc[0:N], acc[N:2 * N], acc[2 * N:3 * N], acc[3 * N:]], axis=1)

        at = jnp.maximum(
            jax.lax.dot(pooled_flat.astype(bf16), W_socT,
                        preferred_element_type=f32) + b_soc, 0.0)
        et = jnp.maximum(
            jax.lax.dot(rel_t.astype(bf16), W_posT,
                        preferred_element_type=f32) + b_pos, 0.0)
        x = jnp.concatenate([et, at], axis=1)
        gates = (jax.lax.dot(x.astype(bf16), W_ihT,
                             preferred_element_type=f32)
                 + jax.lax.dot(ht.astype(bf16), W_hhT,
                               preferred_element_type=f32)
                 + b_g)
        i_g = jax.nn.sigmoid(gates[:, 0:HID])
        f_g = jax.nn.sigmoid(gates[:, HID:2 * HID])
        g_g = jnp.tanh(gates[:, 2 * HID:3 * HID])
        o_g = jax.nn.sigmoid(gates[:, 3 * HID:4 * HID])
        c2 = f_g * ct + i_g * g_g
        h2_ = o_g * jnp.tanh(c2)
        return h2_, c2

    ht0 = jnp.broadcast_to(h0_ref[...], (N, HID))
    ct0 = jnp.broadcast_to(c0_ref[...], (N, HID))

    def enc_body(t, carry):
        ht, ct = carry
        return step(ht, ct, trajC_ref[t], trajT_ref[t], relC_ref[t])

    ht, ct = jax.lax.fori_loop(0, T_ENC, enc_body, (ht0, ct0))

    W_predT = W_predT_ref[...].astype(bf16)
    b_pred = b_pred_ref[...]

    def dec_body(t, carry):
        ht, ct, ppoint = carry
        out = jax.lax.dot(ht.astype(bf16), W_predT,
                          preferred_element_type=f32) + b_pred
        out_ref[t] = out
        prel = out[:, 0:2]
        ppoint = ppoint + prel
        traj_ref[t] = ppoint
        rowx = jnp.sum(eye_f * ppoint[:, 0:1], axis=0, keepdims=True)
        rowy = jnp.sum(eye_f * ppoint[:, 1:2], axis=0, keepdims=True)
        prow2 = jnp.concatenate([rowx, rowy], axis=0)
        ht, ct = step(ht, ct, ppoint, prow2, prel)
        return ht, ct, ppoint

    jax.lax.fori_loop(0, T_DEC, dec_body, (ht, ct, trajC_ref[T_ENC - 1]))


def kernel(past_traj, past_traj_rel, past_traj_timestamp_mask, is_predictable,
           same_scene_mask, W_pos, b_pos, W_soc, b_soc, W_ih, W_hh, b_ih,
           b_hh, h0, c0, W_pred, b_pred):
    f32 = jnp.float32
    trajT = jnp.transpose(past_traj, (1, 2, 0))
    trajC = jnp.transpose(past_traj, (1, 0, 2))
    relC = jnp.transpose(past_traj_rel, (1, 0, 2))
    scene_col = same_scene_mask
    scene_row = jnp.reshape(same_scene_mask, (1, N))
    out, traj = pl.pallas_call(
        _social_lstm_body,
        out_shape=[
            jax.ShapeDtypeStruct((T_DEC, N, 5), f32),
            jax.ShapeDtypeStruct((T_DEC, N, 2), f32),
        ],
    )(trajT, trajC, relC, scene_col, scene_row,
      W_pos.T, b_pos[None, :], W_soc.T, b_soc[None, :],
      W_ih.T, W_hh.T, (b_ih + b_hh)[None, :], h0[None, :], c0[None, :],
      W_pred.T, b_pred[None, :])
    return out.transpose(1, 0, 2), traj.transpose(1, 0, 2)


# speedup vs baseline: 42.1363x; 1.0399x over previous
import jax
import jax.numpy as jnp
from jax.experimental import pallas as pl

N = 512
T_ENC = 8
T_DEC = 12
EMBED = 64
HID = 128


def _social_lstm_body(trajT_ref, trajC_ref, relC_ref, scene_col_ref,
                      scene_row_ref, W_posT_ref, b_pos_ref, W_socT_ref,
                      b_soc_ref, W_ihT_ref, W_hhT_ref, b_g_ref, h0_ref,
                      c0_ref, W_predT_ref, b_pred_ref, out_ref, traj_ref):
    f32 = jnp.float32
    bf16 = jnp.bfloat16
    rows_i = jax.lax.broadcasted_iota(jnp.int32, (N, N), 0)
    cols_i = jax.lax.broadcasted_iota(jnp.int32, (N, N), 1)
    noteye = rows_i != cols_i
    eye_f = jnp.where(noteye, 0.0, 1.0).astype(f32)
    tri = jnp.where(rows_i < cols_i, 1.0, 0.0).astype(bf16)
    sn = (scene_col_ref[...] == scene_row_ref[...]) & noteye

    W_posT = W_posT_ref[...].astype(bf16)
    b_pos = b_pos_ref[...]
    W_socT = W_socT_ref[...].astype(bf16)
    b_soc = b_soc_ref[...]
    W_ihT = W_ihT_ref[...].astype(bf16)
    W_hhT = W_hhT_ref[...].astype(bf16)
    b_g = b_g_ref[...]

    def step(ht, ct, pcol, prow2, rel_t):
        rx = prow2[0:1, :] - pcol[:, 0:1]
        ry = prow2[1:2, :] - pcol[:, 1:2]
        within = (rx < 0.99) & (rx > -0.99) & (ry < 0.99) & (ry > -0.99)
        mb = within & sn
        ax = (rx + 1.0) >= 1.0
        ay = (ry + 1.0) >= 1.0
        q = []
        for g in range(4):
            qx = ax if (g // 2) else jnp.logical_not(ax)
            qy = ay if (g % 2) else jnp.logical_not(ay)
            q.append(mb & qx & qy)
        m_all = jnp.concatenate(q, axis=0)
        m_bf = jnp.where(m_all, 1.0, 0.0).astype(bf16)
        rank = jax.lax.dot(m_bf, tri, preferred_element_type=f32)
        rid = jnp.where(m_all, rank, -1.0)
        gmax = (jnp.max(rid) + 1.0).astype(jnp.int32)

        h1 = ht.astype(bf16)
        r1 = ht - h1.astype(f32)
        h2 = r1.astype(bf16)
        r2 = r1 - h2.astype(f32)
        h3 = r2.astype(bf16)
        h_cat = jnp.concatenate([h1, h2, h3], axis=1)

        def one_rank(rf):
            mr = jnp.where(rid == rf, 1.0, 0.0).astype(bf16)
            d = jax.lax.dot(mr, h_cat, preferred_element_type=f32)
            return (d[:, :HID] + d[:, HID:2 * HID]) + d[:, 2 * HID:]

        gmax_f = jnp.max(rid) + 1.0

        def group_body(r, acc):
            rf = (2 * r).astype(f32)
            acc = acc + one_rank(rf)
            u1 = one_rank(rf + 1.0)
            return acc + jnp.where(rf + 1.0 < gmax_f, u1, 0.0)

        acc0 = jnp.zeros((4 * N, HID), f32)
        acc = jax.lax.fori_loop(0, (gmax + 1) // 2, group_body, acc0)
        pooled_flat = jnp.concatenate(
            [acc[0:N], acc[N:2 * N], acc[2 * N:3 * N], acc[3 * N:]], axis=1)

        at = jnp.maximum(
            jax.lax.dot(pooled_flat.astype(bf16), W_socT,
                        preferred_element_type=f32) + b_soc, 0.0)
        et = jnp.maximum(
            jax.lax.dot(rel_t.astype(bf16), W_posT,
                        preferred_element_type=f32) + b_pos, 0.0)
        x = jnp.concatenate([et, at], axis=1)
        gates = (jax.lax.dot(x.astype(bf16), W_ihT,
                             preferred_element_type=f32)
                 + jax.lax.dot(ht.astype(bf16), W_hhT,
                               preferred_element_type=f32)
                 + b_g)
        i_g = jax.nn.sigmoid(gates[:, 0:HID])
        f_g = jax.nn.sigmoid(gates[:, HID:2 * HID])
        g_g = jnp.tanh(gates[:, 2 * HID:3 * HID])
        o_g = jax.nn.sigmoid(gates[:, 3 * HID:4 * HID])
        c2 = f_g * ct + i_g * g_g
        h2_ = o_g * jnp.tanh(c2)
        return h2_, c2

    ht0 = jnp.broadcast_to(h0_ref[...], (N, HID))
    ct0 = jnp.broadcast_to(c0_ref[...], (N, HID))

    def enc_body(t, carry):
        ht, ct = carry
        return step(ht, ct, trajC_ref[t], trajT_ref[t], relC_ref[t])

    ht, ct = jax.lax.fori_loop(0, T_ENC, enc_body, (ht0, ct0))

    W_predT = W_predT_ref[...].astype(bf16)
    b_pred = b_pred_ref[...]

    def dec_body(t, carry):
        ht, ct, ppoint = carry
        out = jax.lax.dot(ht.astype(bf16), W_predT,
                          preferred_element_type=f32) + b_pred
        out_ref[t] = out
        prel = out[:, 0:2]
        ppoint = ppoint + prel
        traj_ref[t] = ppoint
        rowx = jnp.sum(eye_f * ppoint[:, 0:1], axis=0, keepdims=True)
        rowy = jnp.sum(eye_f * ppoint[:, 1:2], axis=0, keepdims=True)
        prow2 = jnp.concatenate([rowx, rowy], axis=0)
        ht, ct = step(ht, ct, ppoint, prow2, prel)
        return ht, ct, ppoint

    jax.lax.fori_loop(0, T_DEC, dec_body, (ht, ct, trajC_ref[T_ENC - 1]))


def kernel(past_traj, past_traj_rel, past_traj_timestamp_mask, is_predictable,
           same_scene_mask, W_pos, b_pos, W_soc, b_soc, W_ih, W_hh, b_ih,
           b_hh, h0, c0, W_pred, b_pred):
    f32 = jnp.float32
    trajT = jnp.transpose(past_traj, (1, 2, 0))
    trajC = jnp.transpose(past_traj, (1, 0, 2))
    relC = jnp.transpose(past_traj_rel, (1, 0, 2))
    scene_col = same_scene_mask
    scene_row = jnp.reshape(same_scene_mask, (1, N))
    out, traj = pl.pallas_call(
        _social_lstm_body,
        out_shape=[
            jax.ShapeDtypeStruct((T_DEC, N, 5), f32),
            jax.ShapeDtypeStruct((T_DEC, N, 2), f32),
        ],
    )(trajT, trajC, relC, scene_col, scene_row,
      W_pos.T, b_pos[None, :], W_soc.T, b_soc[None, :],
      W_ih.T, W_hh.T, (b_ih + b_hh)[None, :], h0[None, :], c0[None, :],
      W_pred.T, b_pred[None, :])
    return out.transpose(1, 0, 2), traj.transpose(1, 0, 2)
